# baseline (speedup 1.0000x reference)
import functools

import jax
import jax.numpy as jnp
from jax.experimental import pallas as pl
from jax.experimental.pallas import tpu as pltpu


_NSPLIT = 2


def _gcn_kernel(seq_ref, w_ref, *rest):
    adj_refs = rest[:_NSPLIT]
    bias_ref, alpha_ref, out_ref, fts_ref = rest[_NSPLIT:]

    @pl.when(pl.program_id(0) == 0)
    def _():
        fts_ref[...] = jax.lax.dot_general(
            seq_ref[...], w_ref[...],
            dimension_numbers=(((1,), (1,)), ((), ())),
            preferred_element_type=jnp.float32,
        )

    subtm = adj_refs[0].shape[0]
    for s, adj_ref in enumerate(adj_refs):
        acc = jax.lax.dot_general(
            adj_ref[...], fts_ref[...],
            dimension_numbers=(((1,), (0,)), ((), ())),
            preferred_element_type=jnp.float32,
        )
        acc = acc + bias_ref[...]
        out_ref[s * subtm:(s + 1) * subtm, :] = jnp.where(
            acc >= 0, acc, alpha_ref[0, 0] * acc)


def _adj_spec(subtm, n, s):
    return pl.BlockSpec((subtm, n), lambda i, s=s: (i * _NSPLIT + s, 0))


@functools.partial(jax.jit, static_argnames=("interpret",))
def _gcn(seq2d, adj2d, W, bias2d, alpha2d, interpret=False):
    n, din = seq2d.shape
    dout = W.shape[0]
    tm = 400 if n % 400 == 0 else n
    grid = (n // tm,)
    subtm = tm // _NSPLIT

    out = pl.pallas_call(
        _gcn_kernel,
        grid=grid,
        in_specs=[
            pl.BlockSpec((n, din), lambda i: (0, 0)),
            pl.BlockSpec((dout, din), lambda i: (0, 0)),
        ] + [
            _adj_spec(subtm, n, s) for s in range(_NSPLIT)
        ] + [
            pl.BlockSpec((1, dout), lambda i: (0, 0)),
            pl.BlockSpec((1, 1), lambda i: (0, 0)),
        ],
        out_specs=pl.BlockSpec((tm, dout), lambda i: (i, 0)),
        out_shape=jax.ShapeDtypeStruct((n, dout), jnp.float32),
        scratch_shapes=[pltpu.VMEM((n, dout), jnp.float32)],
        compiler_params=pltpu.CompilerParams(
            dimension_semantics=("arbitrary",),
        ),
        interpret=interpret,
    )(seq2d, W, *([adj2d] * _NSPLIT), bias2d, alpha2d)
    return out


def kernel(seq, adj, W, bias, alpha):
    b, n, din = seq.shape
    dout = W.shape[0]
    seq2d = seq.reshape(n, din)
    adj2d = adj.reshape(n, n)
    bias2d = bias.reshape(1, dout)
    alpha2d = alpha.reshape(1, 1)
    out = _gcn(seq2d, adj2d, W, bias2d, alpha2d)
    return out.reshape(b, n, dout)
